# baseline (speedup 1.0000x reference)
import functools

import jax
import jax.numpy as jnp
from jax import lax
from jax.experimental import pallas as pl
from jax.experimental.pallas import tpu as pltpu
from jax.experimental.pallas import tpu_sc as plsc

N = 10000
D = 256
DH = 128
E = 160000
NL = 2
SLOPE = 0.2

NC, NS = 2, 16
NROWS = 10240
EPAD = 163840
NCHUNK = EPAD // 128
CPS_EDGE = NCHUNK // NS
CPS_DEG = NCHUNK // (NC * NS)
ROWS_PS = NROWS // NS

_f32 = jnp.float32
_i32 = jnp.int32


def _leaky(x):
    return jnp.where(x >= 0, x, SLOPE * x)


def _prep_body(wpd, wpm, owd, wvd, bvd, obd, owm, wvm, bvm, obm,
               adt, amt, bd, bm):
    dn_mm = (((1,), (0,)), ((), ()))
    md = lax.dot_general(owd[...], wvd[...], dn_mm, preferred_element_type=_f32)
    mm = lax.dot_general(owm[...], wvm[...], dn_mm, preferred_element_type=_f32)
    dn_tt = (((0,), (1,)), ((), ()))
    adt[...] = lax.dot_general(wpd[...], md, dn_tt, preferred_element_type=_f32)
    amt[...] = lax.dot_general(wpm[...], mm, dn_tt, preferred_element_type=_f32)
    dn_bt = (((1,), (1,)), ((), ()))
    bd[...] = lax.dot_general(bvd[...], owd[...], dn_bt,
                              preferred_element_type=_f32) + obd[...]
    bm[...] = lax.dot_general(bvm[...], owm[...], dn_bt,
                              preferred_element_type=_f32) + obm[...]


def _prep(wpd, wpm, owd, wvd, bvd, obd, owm, wvm, bvm, obm):
    kd, km = wpd.shape[1], wpm.shape[1]
    return pl.pallas_call(
        _prep_body,
        out_shape=[
            jax.ShapeDtypeStruct((kd, D), _f32),
            jax.ShapeDtypeStruct((km, D), _f32),
            jax.ShapeDtypeStruct((1, D), _f32),
            jax.ShapeDtypeStruct((1, D), _f32),
        ],
    )(wpd, wpm, owd, wvd, bvd, obd, owm, wvm, bvm, obm)


def _proj_body(d, m, nt, adt, amt, bd, bm, e2):
    dn = (((1,), (0,)), ((), ()))
    ed = lax.dot_general(d[...], adt[...], dn, preferred_element_type=_f32) + bd[...]
    em = lax.dot_general(m[...], amt[...], dn, preferred_element_type=_f32) + bm[...]
    e = jnp.where(nt[...] == 1, ed, em)
    e2[0] = e[:, :DH]
    e2[1] = e[:, DH:]


def _proj(d_features, m_features, nt2, adt, amt, bd, bm):
    blk = 256
    kd, km = d_features.shape[1], m_features.shape[1]
    grid = (pl.cdiv(N, blk),)
    return pl.pallas_call(
        _proj_body,
        grid=grid,
        in_specs=[
            pl.BlockSpec((blk, kd), lambda i: (i, 0)),
            pl.BlockSpec((blk, km), lambda i: (i, 0)),
            pl.BlockSpec((blk, 1), lambda i: (i, 0)),
            pl.BlockSpec((kd, D), lambda i: (0, 0)),
            pl.BlockSpec((km, D), lambda i: (0, 0)),
            pl.BlockSpec((1, D), lambda i: (0, 0)),
            pl.BlockSpec((1, D), lambda i: (0, 0)),
        ],
        out_specs=pl.BlockSpec((NC, blk, DH), lambda i: (0, i, 0)),
        out_shape=jax.ShapeDtypeStruct((NC, N, DH), _f32),
    )(d_features, m_features, nt2, adt, amt, bd, bm)


def _dense_body(e2, m2, dg2, wst, wnt, wrt, bs, bn, br, y2):
    x = jnp.concatenate([e2[0], e2[1]], axis=1)
    msg = jnp.concatenate([m2[0], m2[1]], axis=1)
    deg = dg2[0] + dg2[1]
    scale = 1.0 / (jnp.maximum(deg, 1.0) * jnp.maximum(deg, 1e-6))
    dn = (((1,), (0,)), ((), ()))
    self_h = _leaky(
        lax.dot_general(x, wst[...], dn, preferred_element_type=_f32) + bs[...])
    neigh_h = _leaky(
        lax.dot_general(msg * scale, wnt[...], dn,
                        preferred_element_type=_f32) + bn[...])
    y = _leaky(
        lax.dot_general(self_h + neigh_h, wrt[...], dn,
                        preferred_element_type=_f32) + br[...])
    y2[0] = y[:, :DH]
    y2[1] = y[:, DH:]


def _dense(e2, m2, dg2, wst, wnt, wrt, bs, bn, br):
    blk = 256
    grid = (pl.cdiv(N, blk),)
    fullspec = lambda a, b: pl.BlockSpec((a, b), lambda i: (0, 0))
    return pl.pallas_call(
        _dense_body,
        grid=grid,
        in_specs=[
            pl.BlockSpec((NC, blk, DH), lambda i: (0, i, 0)),
            pl.BlockSpec((NC, blk, DH), lambda i: (0, i, 0)),
            pl.BlockSpec((NC, blk, 1), lambda i: (0, i, 0)),
            fullspec(D, D), fullspec(D, D), fullspec(D, D),
            fullspec(1, D), fullspec(1, D), fullspec(1, D),
        ],
        out_specs=pl.BlockSpec((NC, blk, DH), lambda i: (0, i, 0)),
        out_shape=jax.ShapeDtypeStruct((NC, N, DH), _f32),
    )(e2, m2, dg2, wst, wnt, wrt, bs, bn, br)


_MESH = plsc.VectorSubcoreMesh(core_axis_name="c", subcore_axis_name="s",
                               num_cores=NC, num_subcores=NS)


def _deg_body(esd, ones_h, zrow, deg2, idx_all, ones_v, dacc, sem_i, sem_s):
    c = lax.axis_index("c")
    s = lax.axis_index("s")
    for z in range(ROWS_PS // 128):
        pltpu.sync_copy(zrow, dacc.at[pl.ds(s * ROWS_PS + z * 128, 128)])
    base = c * (NCHUNK // NC) + s * CPS_DEG
    pltpu.async_copy(esd.at[pl.ds(base, CPS_DEG)], idx_all, sem_i)
    pltpu.sync_copy(ones_h, ones_v)
    pltpu.make_async_copy(esd.at[pl.ds(base, CPS_DEG)], idx_all, sem_i).wait()
    plsc.subcore_barrier()

    GRP = 8

    def deg_step(g, carry):
        for k in range(GRP):
            pltpu.async_copy(ones_v, dacc.at[idx_all.at[g * GRP + k].at[1]],
                             sem_s, add=True)
        for k in range(GRP):
            pltpu.make_async_copy(ones_v,
                                  dacc.at[idx_all.at[g * GRP + k].at[1]],
                                  sem_s).wait()
        return carry

    lax.fori_loop(0, CPS_DEG // GRP, deg_step, 0)

    plsc.subcore_barrier()
    pltpu.sync_copy(dacc.at[pl.ds(s * ROWS_PS, ROWS_PS)],
                    deg2.at[c].at[pl.ds(s * ROWS_PS, ROWS_PS)])


_deg = functools.partial(
    pl.kernel,
    out_type=jax.ShapeDtypeStruct((NC, NROWS, DH), _f32),
    mesh=_MESH,
    scratch_types=[
        pltpu.VMEM((CPS_DEG, 2, 128), _i32),
        pltpu.VMEM((128, DH), _f32),
        pltpu.VMEM_SHARED((NROWS, DH), _f32),
        pltpu.SemaphoreType.DMA,
        pltpu.SemaphoreType.DMA,
    ],
)(_deg_body)


def _train_body(st2, dt2, e2, si2, di2, tidx, rows_v, sem):
    c = lax.axis_index("c")
    s = lax.axis_index("s")

    def gather_train(t2, out):
        for t in range(2):
            pltpu.sync_copy(t2.at[s * 2 + t], tidx)
            pltpu.async_copy(e2.at[c].at[tidx.at[0]], rows_v, sem).wait()
            pltpu.sync_copy(rows_v,
                            out.at[c].at[pl.ds(s * 256 + t * 128, 128)])

    gather_train(st2, si2)
    gather_train(dt2, di2)


_train = functools.partial(
    pl.kernel,
    out_type=(
        jax.ShapeDtypeStruct((NC, 4096, DH), _f32),
        jax.ShapeDtypeStruct((NC, 4096, DH), _f32),
    ),
    mesh=_MESH,
    scratch_types=[
        pltpu.VMEM((1, 128), _i32),
        pltpu.VMEM((128, DH), _f32),
        pltpu.SemaphoreType.DMA,
    ],
)(_train_body)


def _edge_body(esd, e2, zrow, m2, idx_all, rows_a, rows_b, acc,
               sem_a, sem_b, sem_sa, sem_sb, sem_i):
    c = lax.axis_index("c")
    s = lax.axis_index("s")
    STG = CPS_EDGE // 2
    pltpu.async_copy(esd.at[pl.ds(s * CPS_EDGE, STG)], idx_all, sem_i)
    for z in range(ROWS_PS // 128):
        pltpu.sync_copy(zrow, acc.at[pl.ds(s * ROWS_PS + z * 128, 128)])
    pltpu.make_async_copy(esd.at[pl.ds(s * CPS_EDGE, STG)], idx_all,
                          sem_i).wait()
    plsc.subcore_barrier()

    tab = e2.at[c]

    def run_stage(stg, carry):
        half = STG // 2
        pltpu.async_copy(tab.at[idx_all.at[0].at[0]], rows_a, sem_a)
        pltpu.async_copy(tab.at[idx_all.at[1].at[0]], rows_b, sem_b)

        def step(j, carry2):
            pltpu.make_async_copy(tab.at[idx_all.at[2 * j].at[0]],
                                  rows_a, sem_a).wait()
            pltpu.sync_copy(rows_a, acc.at[idx_all.at[2 * j].at[1]],
                            add=True)

            @pl.when(j < half - 1)
            def _():
                pltpu.async_copy(tab.at[idx_all.at[2 * j + 2].at[0]],
                                 rows_a, sem_a)

            pltpu.make_async_copy(tab.at[idx_all.at[2 * j + 1].at[0]],
                                  rows_b, sem_b).wait()
            pltpu.sync_copy(rows_b, acc.at[idx_all.at[2 * j + 1].at[1]],
                            add=True)

            @pl.when(j < half - 1)
            def _():
                pltpu.async_copy(tab.at[idx_all.at[2 * j + 3].at[0]],
                                 rows_b, sem_b)

            return carry2

        lax.fori_loop(0, half, step, 0)

        @pl.when(stg == 0)
        def _():
            pltpu.sync_copy(esd.at[pl.ds(s * CPS_EDGE + STG, STG)], idx_all)

        return carry

    lax.fori_loop(0, 2, run_stage, 0)
    plsc.subcore_barrier()
    pltpu.sync_copy(acc.at[pl.ds(s * ROWS_PS, ROWS_PS)],
                    m2.at[c].at[pl.ds(s * ROWS_PS, ROWS_PS)])


_edge = functools.partial(
    pl.kernel,
    out_type=jax.ShapeDtypeStruct((NC, NROWS, DH), _f32),
    mesh=_MESH,
    scratch_types=[
        pltpu.VMEM((CPS_EDGE // 2, 2, 128), _i32),
        pltpu.VMEM((128, DH), _f32),
        pltpu.VMEM((128, DH), _f32),
        pltpu.VMEM_SHARED((NROWS, DH), _f32),
        pltpu.SemaphoreType.DMA,
        pltpu.SemaphoreType.DMA,
        pltpu.SemaphoreType.DMA,
        pltpu.SemaphoreType.DMA,
        pltpu.SemaphoreType.DMA,
    ],
)(_edge_body)


def kernel(d_features, m_features, node_type, edge_index, src_train, dst_train,
           W_proj_d, W_proj_m, attn_d_in_w, attn_d_in_b, attn_d_out_w,
           attn_d_out_b, attn_m_in_w, attn_m_in_b, attn_m_out_w, attn_m_out_b,
           W_self, b_self, W_neigh, b_neigh, W_res, b_res):
    nt2 = node_type.astype(_i32).reshape(N, 1)
    src = edge_index[0].astype(_i32)
    dst = edge_index[1].astype(_i32)
    npad = EPAD - E
    pad_ar = jnp.arange(npad, dtype=_i32)
    src_p = jnp.concatenate([src, pad_ar % N])
    dst_p = jnp.concatenate([dst, N + pad_ar % (NROWS - N)])
    esd = jnp.stack([src_p.reshape(NCHUNK, 128), dst_p.reshape(NCHUNK, 128)],
                    axis=1)
    st2 = src_train.astype(_i32).reshape(32, 1, 128)
    dt2 = dst_train.astype(_i32).reshape(32, 1, 128)

    zrow = jnp.zeros((128, DH), _f32)
    ones_h = jnp.ones((128, DH), _f32)
    deg2 = _deg(esd, ones_h, zrow)
    dgs = deg2[:, :, :1]

    wvd = attn_d_in_w[2 * D:]
    bvd = attn_d_in_b[2 * D:].reshape(1, D)
    wvm = attn_m_in_w[2 * D:]
    bvm = attn_m_in_b[2 * D:].reshape(1, D)
    adt, amt, bd, bm = _prep(W_proj_d, W_proj_m,
                             attn_d_out_w, wvd, bvd, attn_d_out_b.reshape(1, D),
                             attn_m_out_w, wvm, bvm, attn_m_out_b.reshape(1, D))

    e2 = _proj(d_features, m_features, nt2, adt, amt, bd, bm)

    si2, di2 = _train(st2, dt2, e2)

    for l in range(NL):
        m2 = _edge(esd, e2, zrow)
        e2 = _dense(e2, m2, dgs,
                    W_self[l].T, W_neigh[l].T, W_res[l].T,
                    b_self[l].reshape(1, D), b_neigh[l].reshape(1, D),
                    b_res[l].reshape(1, D))

    emb = jnp.concatenate([e2[0], e2[1]], axis=1)
    src_init = jnp.concatenate([si2[0], si2[1]], axis=1)
    dst_init = jnp.concatenate([di2[0], di2[1]], axis=1)
    return emb, src_init, dst_init


# speedup vs baseline: 1.0186x; 1.0186x over previous
import functools

import jax
import jax.numpy as jnp
from jax import lax
from jax.experimental import pallas as pl
from jax.experimental.pallas import tpu as pltpu
from jax.experimental.pallas import tpu_sc as plsc

N = 10000
D = 256
DH = 128
E = 160000
NL = 2
SLOPE = 0.2

NC, NS = 2, 16
NROWS = 10240
EPAD = 163840
NCHUNK = EPAD // 128
CPS_EDGE = NCHUNK // NS
CPS_DEG = NCHUNK // (NC * NS)
ROWS_PS = NROWS // NS

_f32 = jnp.float32
_i32 = jnp.int32


def _leaky(x):
    return jnp.where(x >= 0, x, SLOPE * x)


def _prep_body(wpd, wpm, owd, wvd, bvd, obd, owm, wvm, bvm, obm,
               adt, amt, bd, bm):
    dn_mm = (((1,), (0,)), ((), ()))
    md = lax.dot_general(owd[...], wvd[...], dn_mm, preferred_element_type=_f32)
    mm = lax.dot_general(owm[...], wvm[...], dn_mm, preferred_element_type=_f32)
    dn_tt = (((0,), (1,)), ((), ()))
    adt[...] = lax.dot_general(wpd[...], md, dn_tt, preferred_element_type=_f32)
    amt[...] = lax.dot_general(wpm[...], mm, dn_tt, preferred_element_type=_f32)
    dn_bt = (((1,), (1,)), ((), ()))
    bd[...] = lax.dot_general(bvd[...], owd[...], dn_bt,
                              preferred_element_type=_f32) + obd[...]
    bm[...] = lax.dot_general(bvm[...], owm[...], dn_bt,
                              preferred_element_type=_f32) + obm[...]


def _prep(wpd, wpm, owd, wvd, bvd, obd, owm, wvm, bvm, obm):
    kd, km = wpd.shape[1], wpm.shape[1]
    return pl.pallas_call(
        _prep_body,
        out_shape=[
            jax.ShapeDtypeStruct((kd, D), _f32),
            jax.ShapeDtypeStruct((km, D), _f32),
            jax.ShapeDtypeStruct((1, D), _f32),
            jax.ShapeDtypeStruct((1, D), _f32),
        ],
    )(wpd, wpm, owd, wvd, bvd, obd, owm, wvm, bvm, obm)


def _proj_body(d, m, nt, adt, amt, bd, bm, e2):
    dn = (((1,), (0,)), ((), ()))
    ed = lax.dot_general(d[...], adt[...], dn, preferred_element_type=_f32) + bd[...]
    em = lax.dot_general(m[...], amt[...], dn, preferred_element_type=_f32) + bm[...]
    e = jnp.where(nt[...] == 1, ed, em)
    e2[0] = e[:, :DH]
    e2[1] = e[:, DH:]


def _proj(d_features, m_features, nt2, adt, amt, bd, bm):
    blk = 256
    kd, km = d_features.shape[1], m_features.shape[1]
    grid = (pl.cdiv(N, blk),)
    return pl.pallas_call(
        _proj_body,
        grid=grid,
        in_specs=[
            pl.BlockSpec((blk, kd), lambda i: (i, 0)),
            pl.BlockSpec((blk, km), lambda i: (i, 0)),
            pl.BlockSpec((blk, 1), lambda i: (i, 0)),
            pl.BlockSpec((kd, D), lambda i: (0, 0)),
            pl.BlockSpec((km, D), lambda i: (0, 0)),
            pl.BlockSpec((1, D), lambda i: (0, 0)),
            pl.BlockSpec((1, D), lambda i: (0, 0)),
        ],
        out_specs=pl.BlockSpec((NC, blk, DH), lambda i: (0, i, 0)),
        out_shape=jax.ShapeDtypeStruct((NC, N, DH), _f32),
    )(d_features, m_features, nt2, adt, amt, bd, bm)


def _dense_math(e2, m2, dg2, wst, wnt, wrt, bs, bn, br):
    x = jnp.concatenate([e2[0], e2[1]], axis=1)
    msg = jnp.concatenate([m2[0], m2[1]], axis=1)
    deg = dg2[0] + dg2[1]
    scale = 1.0 / (jnp.maximum(deg, 1.0) * jnp.maximum(deg, 1e-6))
    dn = (((1,), (0,)), ((), ()))
    self_h = _leaky(
        lax.dot_general(x, wst[...], dn, preferred_element_type=_f32) + bs[...])
    neigh_h = _leaky(
        lax.dot_general(msg * scale, wnt[...], dn,
                        preferred_element_type=_f32) + bn[...])
    return _leaky(
        lax.dot_general(self_h + neigh_h, wrt[...], dn,
                        preferred_element_type=_f32) + br[...])


def _dense_body(e2, m2, dg2, wst, wnt, wrt, bs, bn, br, y2):
    y = _dense_math(e2, m2, dg2, wst, wnt, wrt, bs, bn, br)
    y2[0] = y[:, :DH]
    y2[1] = y[:, DH:]


def _dense_last_body(e2, m2, dg2, wst, wnt, wrt, bs, bn, br, y):
    y[...] = _dense_math(e2, m2, dg2, wst, wnt, wrt, bs, bn, br)


def _dense(e2, m2, dg2, wst, wnt, wrt, bs, bn, br, last=False):
    blk = 256
    grid = (pl.cdiv(N, blk),)
    fullspec = lambda a, b: pl.BlockSpec((a, b), lambda i: (0, 0))
    if last:
        out_specs = pl.BlockSpec((blk, D), lambda i: (i, 0))
        out_shape = jax.ShapeDtypeStruct((N, D), _f32)
        body = _dense_last_body
    else:
        out_specs = pl.BlockSpec((NC, blk, DH), lambda i: (0, i, 0))
        out_shape = jax.ShapeDtypeStruct((NC, N, DH), _f32)
        body = _dense_body
    return pl.pallas_call(
        body,
        grid=grid,
        in_specs=[
            pl.BlockSpec((NC, blk, DH), lambda i: (0, i, 0)),
            pl.BlockSpec((NC, blk, DH), lambda i: (0, i, 0)),
            pl.BlockSpec((NC, blk, 1), lambda i: (0, i, 0)),
            fullspec(D, D), fullspec(D, D), fullspec(D, D),
            fullspec(1, D), fullspec(1, D), fullspec(1, D),
        ],
        out_specs=out_specs,
        out_shape=out_shape,
    )(e2, m2, dg2, wst, wnt, wrt, bs, bn, br)


_MESH = plsc.VectorSubcoreMesh(core_axis_name="c", subcore_axis_name="s",
                               num_cores=NC, num_subcores=NS)


def _deg_body(esd, ones_h, zrow, deg2, idx_all, ones_v, dacc, sem_i, sem_s):
    c = lax.axis_index("c")
    s = lax.axis_index("s")
    for z in range(ROWS_PS // 128):
        pltpu.sync_copy(zrow, dacc.at[pl.ds(s * ROWS_PS + z * 128, 128)])
    base = c * (NCHUNK // NC) + s * CPS_DEG
    pltpu.async_copy(esd.at[pl.ds(base, CPS_DEG)], idx_all, sem_i)
    pltpu.sync_copy(ones_h, ones_v)
    pltpu.make_async_copy(esd.at[pl.ds(base, CPS_DEG)], idx_all, sem_i).wait()
    plsc.subcore_barrier()

    GRP = 8

    def deg_step(g, carry):
        for k in range(GRP):
            pltpu.async_copy(ones_v, dacc.at[idx_all.at[g * GRP + k].at[1]],
                             sem_s, add=True)
        for k in range(GRP):
            pltpu.make_async_copy(ones_v,
                                  dacc.at[idx_all.at[g * GRP + k].at[1]],
                                  sem_s).wait()
        return carry

    lax.fori_loop(0, CPS_DEG // GRP, deg_step, 0)

    plsc.subcore_barrier()
    pltpu.sync_copy(dacc.at[pl.ds(s * ROWS_PS, ROWS_PS)],
                    deg2.at[c].at[pl.ds(s * ROWS_PS, ROWS_PS)])


_deg = functools.partial(
    pl.kernel,
    out_type=jax.ShapeDtypeStruct((NC, NROWS, DH), _f32),
    mesh=_MESH,
    scratch_types=[
        pltpu.VMEM((CPS_DEG, 2, 128), _i32),
        pltpu.VMEM((128, DH), _f32),
        pltpu.VMEM_SHARED((NROWS, DH), _f32),
        pltpu.SemaphoreType.DMA,
        pltpu.SemaphoreType.DMA,
    ],
)(_deg_body)


def _train_body(st2, dt2, e2, si3, di3, tidx, rows_v, sem):
    c = lax.axis_index("c")
    s = lax.axis_index("s")

    def gather_train(t2, out):
        for t in range(2):
            pltpu.sync_copy(t2.at[s * 2 + t], tidx)
            pltpu.async_copy(e2.at[c].at[tidx.at[0]], rows_v, sem).wait()
            pltpu.sync_copy(rows_v,
                            out.at[pl.ds(s * 256 + t * 128, 128), c])

    gather_train(st2, si3)
    gather_train(dt2, di3)


_train = functools.partial(
    pl.kernel,
    out_type=(
        jax.ShapeDtypeStruct((4096, NC, DH), _f32),
        jax.ShapeDtypeStruct((4096, NC, DH), _f32),
    ),
    mesh=_MESH,
    scratch_types=[
        pltpu.VMEM((1, 128), _i32),
        pltpu.VMEM((128, DH), _f32),
        pltpu.SemaphoreType.DMA,
    ],
)(_train_body)


def _edge_body(esd, e2, zrow, m2, idx_all, rows_a, rows_b, acc,
               sem_a, sem_b, sem_sa, sem_sb, sem_i):
    c = lax.axis_index("c")
    s = lax.axis_index("s")
    STG = CPS_EDGE // 2
    pltpu.async_copy(esd.at[pl.ds(s * CPS_EDGE, STG)], idx_all, sem_i)
    for z in range(ROWS_PS // 128):
        pltpu.sync_copy(zrow, acc.at[pl.ds(s * ROWS_PS + z * 128, 128)])
    pltpu.make_async_copy(esd.at[pl.ds(s * CPS_EDGE, STG)], idx_all,
                          sem_i).wait()
    plsc.subcore_barrier()

    tab = e2.at[c]

    def run_stage(stg, carry):
        half = STG // 2
        pltpu.async_copy(tab.at[idx_all.at[0].at[0]], rows_a, sem_a)
        pltpu.async_copy(tab.at[idx_all.at[1].at[0]], rows_b, sem_b)

        def step(j, carry2):
            pltpu.make_async_copy(tab.at[idx_all.at[2 * j].at[0]],
                                  rows_a, sem_a).wait()
            pltpu.sync_copy(rows_a, acc.at[idx_all.at[2 * j].at[1]],
                            add=True)

            @pl.when(j < half - 1)
            def _():
                pltpu.async_copy(tab.at[idx_all.at[2 * j + 2].at[0]],
                                 rows_a, sem_a)

            pltpu.make_async_copy(tab.at[idx_all.at[2 * j + 1].at[0]],
                                  rows_b, sem_b).wait()
            pltpu.sync_copy(rows_b, acc.at[idx_all.at[2 * j + 1].at[1]],
                            add=True)

            @pl.when(j < half - 1)
            def _():
                pltpu.async_copy(tab.at[idx_all.at[2 * j + 3].at[0]],
                                 rows_b, sem_b)

            return carry2

        lax.fori_loop(0, half, step, 0)

        @pl.when(stg == 0)
        def _():
            pltpu.sync_copy(esd.at[pl.ds(s * CPS_EDGE + STG, STG)], idx_all)

        return carry

    lax.fori_loop(0, 2, run_stage, 0)
    plsc.subcore_barrier()
    pltpu.sync_copy(acc.at[pl.ds(s * ROWS_PS, ROWS_PS)],
                    m2.at[c].at[pl.ds(s * ROWS_PS, ROWS_PS)])


_edge = functools.partial(
    pl.kernel,
    out_type=jax.ShapeDtypeStruct((NC, NROWS, DH), _f32),
    mesh=_MESH,
    scratch_types=[
        pltpu.VMEM((CPS_EDGE // 2, 2, 128), _i32),
        pltpu.VMEM((128, DH), _f32),
        pltpu.VMEM((128, DH), _f32),
        pltpu.VMEM_SHARED((NROWS, DH), _f32),
        pltpu.SemaphoreType.DMA,
        pltpu.SemaphoreType.DMA,
        pltpu.SemaphoreType.DMA,
        pltpu.SemaphoreType.DMA,
        pltpu.SemaphoreType.DMA,
    ],
)(_edge_body)


def kernel(d_features, m_features, node_type, edge_index, src_train, dst_train,
           W_proj_d, W_proj_m, attn_d_in_w, attn_d_in_b, attn_d_out_w,
           attn_d_out_b, attn_m_in_w, attn_m_in_b, attn_m_out_w, attn_m_out_b,
           W_self, b_self, W_neigh, b_neigh, W_res, b_res):
    nt2 = node_type.astype(_i32).reshape(N, 1)
    src = edge_index[0].astype(_i32)
    dst = edge_index[1].astype(_i32)
    npad = EPAD - E
    pad_ar = jnp.arange(npad, dtype=_i32)
    src_p = jnp.concatenate([src, pad_ar % N])
    dst_p = jnp.concatenate([dst, N + pad_ar % (NROWS - N)])
    esd = jnp.stack([src_p.reshape(NCHUNK, 128), dst_p.reshape(NCHUNK, 128)],
                    axis=1)
    st2 = src_train.astype(_i32).reshape(32, 1, 128)
    dt2 = dst_train.astype(_i32).reshape(32, 1, 128)

    zrow = jnp.zeros((128, DH), _f32)
    ones_h = jnp.ones((128, DH), _f32)
    deg2 = _deg(esd, ones_h, zrow)
    dgs = deg2[:, :, :1]

    wvd = attn_d_in_w[2 * D:]
    bvd = attn_d_in_b[2 * D:].reshape(1, D)
    wvm = attn_m_in_w[2 * D:]
    bvm = attn_m_in_b[2 * D:].reshape(1, D)
    adt, amt, bd, bm = _prep(W_proj_d, W_proj_m,
                             attn_d_out_w, wvd, bvd, attn_d_out_b.reshape(1, D),
                             attn_m_out_w, wvm, bvm, attn_m_out_b.reshape(1, D))

    e2 = _proj(d_features, m_features, nt2, adt, amt, bd, bm)

    si3, di3 = _train(st2, dt2, e2)

    emb = None
    for l in range(NL):
        m2 = _edge(esd, e2, zrow)
        out = _dense(e2, m2, dgs,
                     W_self[l].T, W_neigh[l].T, W_res[l].T,
                     b_self[l].reshape(1, D), b_neigh[l].reshape(1, D),
                     b_res[l].reshape(1, D), last=(l == NL - 1))
        if l == NL - 1:
            emb = out
        else:
            e2 = out

    src_init = si3.reshape(4096, D)
    dst_init = di3.reshape(4096, D)
    return emb, src_init, dst_init


# speedup vs baseline: 1.0982x; 1.0782x over previous
import functools

import jax
import jax.numpy as jnp
from jax import lax
from jax.experimental import pallas as pl
from jax.experimental.pallas import tpu as pltpu
from jax.experimental.pallas import tpu_sc as plsc

N = 10000
D = 256
DH = 128
E = 160000
NL = 2
SLOPE = 0.2

NC, NS = 2, 16
NROWS = 10240
EPAD = 163840
NCHUNK = EPAD // 128
CPS_EDGE = NCHUNK // NS
CPS_DEG = NCHUNK // (NC * NS)
ROWS_PS = NROWS // NS

_f32 = jnp.float32
_i32 = jnp.int32


def _leaky(x):
    return jnp.where(x >= 0, x, SLOPE * x)


def _prep_body(wpd, wpm, owd, wvd, bvd, obd, owm, wvm, bvm, obm,
               adt, amt, bd, bm):
    dn_mm = (((1,), (0,)), ((), ()))
    md = lax.dot_general(owd[...], wvd[...], dn_mm, preferred_element_type=_f32)
    mm = lax.dot_general(owm[...], wvm[...], dn_mm, preferred_element_type=_f32)
    dn_tt = (((0,), (1,)), ((), ()))
    adt[...] = lax.dot_general(wpd[...], md, dn_tt, preferred_element_type=_f32)
    amt[...] = lax.dot_general(wpm[...], mm, dn_tt, preferred_element_type=_f32)
    dn_bt = (((1,), (1,)), ((), ()))
    bd[...] = lax.dot_general(bvd[...], owd[...], dn_bt,
                              preferred_element_type=_f32) + obd[...]
    bm[...] = lax.dot_general(bvm[...], owm[...], dn_bt,
                              preferred_element_type=_f32) + obm[...]


def _prep(wpd, wpm, owd, wvd, bvd, obd, owm, wvm, bvm, obm):
    kd, km = wpd.shape[1], wpm.shape[1]
    return pl.pallas_call(
        _prep_body,
        out_shape=[
            jax.ShapeDtypeStruct((kd, D), _f32),
            jax.ShapeDtypeStruct((km, D), _f32),
            jax.ShapeDtypeStruct((1, D), _f32),
            jax.ShapeDtypeStruct((1, D), _f32),
        ],
    )(wpd, wpm, owd, wvd, bvd, obd, owm, wvm, bvm, obm)


def _proj_body(d, m, nt, adt, amt, bd, bm, e2):
    dn = (((1,), (0,)), ((), ()))
    ed = lax.dot_general(d[...], adt[...], dn, preferred_element_type=_f32) + bd[...]
    em = lax.dot_general(m[...], amt[...], dn, preferred_element_type=_f32) + bm[...]
    e = jnp.where(nt[...] == 1, ed, em)
    e2[0] = e[:, :DH]
    e2[1] = e[:, DH:]


def _proj(d_features, m_features, nt2, adt, amt, bd, bm):
    blk = 512
    kd, km = d_features.shape[1], m_features.shape[1]
    grid = (pl.cdiv(N, blk),)
    return pl.pallas_call(
        _proj_body,
        grid=grid,
        in_specs=[
            pl.BlockSpec((blk, kd), lambda i: (i, 0)),
            pl.BlockSpec((blk, km), lambda i: (i, 0)),
            pl.BlockSpec((blk, 1), lambda i: (i, 0)),
            pl.BlockSpec((kd, D), lambda i: (0, 0)),
            pl.BlockSpec((km, D), lambda i: (0, 0)),
            pl.BlockSpec((1, D), lambda i: (0, 0)),
            pl.BlockSpec((1, D), lambda i: (0, 0)),
        ],
        out_specs=pl.BlockSpec((NC, blk, DH), lambda i: (0, i, 0)),
        out_shape=jax.ShapeDtypeStruct((NC, N, DH), _f32),
    )(d_features, m_features, nt2, adt, amt, bd, bm)


def _dense_math(e2, m2, dg2, wst, wnt, wrt, bs, bn, br):
    x = jnp.concatenate([e2[0], e2[1]], axis=1)
    msg = jnp.concatenate([m2[0], m2[1]], axis=1)
    deg = dg2[0] + dg2[1]
    scale = 1.0 / (jnp.maximum(deg, 1.0) * jnp.maximum(deg, 1e-6))
    dn = (((1,), (0,)), ((), ()))
    self_h = _leaky(
        lax.dot_general(x, wst[...], dn, preferred_element_type=_f32) + bs[...])
    neigh_h = _leaky(
        lax.dot_general(msg * scale, wnt[...], dn,
                        preferred_element_type=_f32) + bn[...])
    return _leaky(
        lax.dot_general(self_h + neigh_h, wrt[...], dn,
                        preferred_element_type=_f32) + br[...])


def _dense_body(e2, m2, dg2, wst, wnt, wrt, bs, bn, br, y2):
    y = _dense_math(e2, m2, dg2, wst, wnt, wrt, bs, bn, br)
    y2[0] = y[:, :DH]
    y2[1] = y[:, DH:]


def _dense_last_body(e2, m2, dg2, wst, wnt, wrt, bs, bn, br, y):
    y[...] = _dense_math(e2, m2, dg2, wst, wnt, wrt, bs, bn, br)


def _dense(e2, m2, dg2, wst, wnt, wrt, bs, bn, br, last=False):
    blk = 512
    grid = (pl.cdiv(N, blk),)
    fullspec = lambda a, b: pl.BlockSpec((a, b), lambda i: (0, 0))
    if last:
        out_specs = pl.BlockSpec((blk, D), lambda i: (i, 0))
        out_shape = jax.ShapeDtypeStruct((N, D), _f32)
        body = _dense_last_body
    else:
        out_specs = pl.BlockSpec((NC, blk, DH), lambda i: (0, i, 0))
        out_shape = jax.ShapeDtypeStruct((NC, N, DH), _f32)
        body = _dense_body
    return pl.pallas_call(
        body,
        grid=grid,
        in_specs=[
            pl.BlockSpec((NC, blk, DH), lambda i: (0, i, 0)),
            pl.BlockSpec((NC, blk, DH), lambda i: (0, i, 0)),
            pl.BlockSpec((NC, blk, 1), lambda i: (0, i, 0)),
            fullspec(D, D), fullspec(D, D), fullspec(D, D),
            fullspec(1, D), fullspec(1, D), fullspec(1, D),
        ],
        out_specs=out_specs,
        out_shape=out_shape,
    )(e2, m2, dg2, wst, wnt, wrt, bs, bn, br)


_MESH = plsc.VectorSubcoreMesh(core_axis_name="c", subcore_axis_name="s",
                               num_cores=NC, num_subcores=NS)


def _deg_body(esd, ones_h, zrow, deg2, idx_all, ones_v, dacc, sem_i, sem_s):
    c = lax.axis_index("c")
    s = lax.axis_index("s")
    for z in range(ROWS_PS // 128):
        pltpu.sync_copy(zrow, dacc.at[pl.ds(s * ROWS_PS + z * 128, 128)])
    base = c * (NCHUNK // NC) + s * CPS_DEG
    pltpu.async_copy(esd.at[pl.ds(base, CPS_DEG)], idx_all, sem_i)
    pltpu.sync_copy(ones_h, ones_v)
    pltpu.make_async_copy(esd.at[pl.ds(base, CPS_DEG)], idx_all, sem_i).wait()
    plsc.subcore_barrier()

    GRP = 8

    def deg_step(g, carry):
        for k in range(GRP):
            pltpu.async_copy(ones_v, dacc.at[idx_all.at[g * GRP + k].at[1]],
                             sem_s, add=True)
        for k in range(GRP):
            pltpu.make_async_copy(ones_v,
                                  dacc.at[idx_all.at[g * GRP + k].at[1]],
                                  sem_s).wait()
        return carry

    lax.fori_loop(0, CPS_DEG // GRP, deg_step, 0)

    plsc.subcore_barrier()
    pltpu.sync_copy(dacc.at[pl.ds(s * ROWS_PS, ROWS_PS)],
                    deg2.at[c].at[pl.ds(s * ROWS_PS, ROWS_PS)])


_deg = functools.partial(
    pl.kernel,
    out_type=jax.ShapeDtypeStruct((NC, NROWS, DH), _f32),
    mesh=_MESH,
    scratch_types=[
        pltpu.VMEM((CPS_DEG, 2, 128), _i32),
        pltpu.VMEM((128, DH), _f32),
        pltpu.VMEM_SHARED((NROWS, DH), _f32),
        pltpu.SemaphoreType.DMA,
        pltpu.SemaphoreType.DMA,
    ],
)(_deg_body)


def _train_body(st2, dt2, e2, si3, di3, tidx, rows_v, sem):
    c = lax.axis_index("c")
    s = lax.axis_index("s")

    def gather_train(t2, out):
        for t in range(2):
            pltpu.sync_copy(t2.at[s * 2 + t], tidx)
            pltpu.async_copy(e2.at[c].at[tidx.at[0]], rows_v, sem).wait()
            pltpu.sync_copy(rows_v,
                            out.at[pl.ds(s * 256 + t * 128, 128), c])

    gather_train(st2, si3)
    gather_train(dt2, di3)


_train = functools.partial(
    pl.kernel,
    out_type=(
        jax.ShapeDtypeStruct((4096, NC, DH), _f32),
        jax.ShapeDtypeStruct((4096, NC, DH), _f32),
    ),
    mesh=_MESH,
    scratch_types=[
        pltpu.VMEM((1, 128), _i32),
        pltpu.VMEM((128, DH), _f32),
        pltpu.SemaphoreType.DMA,
    ],
)(_train_body)


def _edge_body(esd, e2, zrow, m2, idx_all, rows_a, rows_b, acc,
               sem_a, sem_b, sem_sa, sem_sb, sem_i):
    c = lax.axis_index("c")
    s = lax.axis_index("s")
    STG = CPS_EDGE // 2
    pltpu.async_copy(esd.at[pl.ds(s * CPS_EDGE, STG)], idx_all, sem_i)
    for z in range(ROWS_PS // 128):
        pltpu.sync_copy(zrow, acc.at[pl.ds(s * ROWS_PS + z * 128, 128)])
    pltpu.make_async_copy(esd.at[pl.ds(s * CPS_EDGE, STG)], idx_all,
                          sem_i).wait()
    plsc.subcore_barrier()

    tab = e2.at[c]

    def run_stage(stg, carry):
        half = STG // 2
        pltpu.async_copy(tab.at[idx_all.at[0].at[0]], rows_a, sem_a)
        pltpu.async_copy(tab.at[idx_all.at[1].at[0]], rows_b, sem_b)

        def step(j, carry2):
            pltpu.make_async_copy(tab.at[idx_all.at[2 * j].at[0]],
                                  rows_a, sem_a).wait()
            pltpu.sync_copy(rows_a, acc.at[idx_all.at[2 * j].at[1]],
                            add=True)

            @pl.when(j < half - 1)
            def _():
                pltpu.async_copy(tab.at[idx_all.at[2 * j + 2].at[0]],
                                 rows_a, sem_a)

            pltpu.make_async_copy(tab.at[idx_all.at[2 * j + 1].at[0]],
                                  rows_b, sem_b).wait()
            pltpu.sync_copy(rows_b, acc.at[idx_all.at[2 * j + 1].at[1]],
                            add=True)

            @pl.when(j < half - 1)
            def _():
                pltpu.async_copy(tab.at[idx_all.at[2 * j + 3].at[0]],
                                 rows_b, sem_b)

            return carry2

        lax.fori_loop(0, half, step, 0)

        @pl.when(stg == 0)
        def _():
            pltpu.sync_copy(esd.at[pl.ds(s * CPS_EDGE + STG, STG)], idx_all)

        return carry

    lax.fori_loop(0, 2, run_stage, 0)
    plsc.subcore_barrier()
    pltpu.sync_copy(acc.at[pl.ds(s * ROWS_PS, ROWS_PS)],
                    m2.at[c].at[pl.ds(s * ROWS_PS, ROWS_PS)])


_edge = functools.partial(
    pl.kernel,
    out_type=jax.ShapeDtypeStruct((NC, NROWS, DH), _f32),
    mesh=_MESH,
    scratch_types=[
        pltpu.VMEM((CPS_EDGE // 2, 2, 128), _i32),
        pltpu.VMEM((128, DH), _f32),
        pltpu.VMEM((128, DH), _f32),
        pltpu.VMEM_SHARED((NROWS, DH), _f32),
        pltpu.SemaphoreType.DMA,
        pltpu.SemaphoreType.DMA,
        pltpu.SemaphoreType.DMA,
        pltpu.SemaphoreType.DMA,
        pltpu.SemaphoreType.DMA,
    ],
)(_edge_body)


def kernel(d_features, m_features, node_type, edge_index, src_train, dst_train,
           W_proj_d, W_proj_m, attn_d_in_w, attn_d_in_b, attn_d_out_w,
           attn_d_out_b, attn_m_in_w, attn_m_in_b, attn_m_out_w, attn_m_out_b,
           W_self, b_self, W_neigh, b_neigh, W_res, b_res):
    nt2 = node_type.astype(_i32).reshape(N, 1)
    src = edge_index[0].astype(_i32)
    dst = edge_index[1].astype(_i32)
    npad = EPAD - E
    pad_ar = jnp.arange(npad, dtype=_i32)
    src_p = jnp.concatenate([src, pad_ar % N])
    dst_p = jnp.concatenate([dst, N + pad_ar % (NROWS - N)])
    esd = jnp.stack([src_p.reshape(NCHUNK, 128), dst_p.reshape(NCHUNK, 128)],
                    axis=1)
    st2 = src_train.astype(_i32).reshape(32, 1, 128)
    dt2 = dst_train.astype(_i32).reshape(32, 1, 128)

    zrow = jnp.zeros((128, DH), _f32)
    ones_h = jnp.ones((128, DH), _f32)
    deg2 = _deg(esd, ones_h, zrow)
    dgs = deg2[:, :, :1]

    wvd = attn_d_in_w[2 * D:]
    bvd = attn_d_in_b[2 * D:].reshape(1, D)
    wvm = attn_m_in_w[2 * D:]
    bvm = attn_m_in_b[2 * D:].reshape(1, D)
    adt, amt, bd, bm = _prep(W_proj_d, W_proj_m,
                             attn_d_out_w, wvd, bvd, attn_d_out_b.reshape(1, D),
                             attn_m_out_w, wvm, bvm, attn_m_out_b.reshape(1, D))

    e2 = _proj(d_features, m_features, nt2, adt, amt, bd, bm)

    si3, di3 = _train(st2, dt2, e2)

    emb = None
    for l in range(NL):
        m2 = _edge(esd, e2, zrow)
        out = _dense(e2, m2, dgs,
                     W_self[l].T, W_neigh[l].T, W_res[l].T,
                     b_self[l].reshape(1, D), b_neigh[l].reshape(1, D),
                     b_res[l].reshape(1, D), last=(l == NL - 1))
        if l == NL - 1:
            emb = out
        else:
            e2 = out

    src_init = si3.reshape(4096, D)
    dst_init = di3.reshape(4096, D)
    return emb, src_init, dst_init


# speedup vs baseline: 1.1212x; 1.0209x over previous
import functools

import jax
import jax.numpy as jnp
from jax import lax
from jax.experimental import pallas as pl
from jax.experimental.pallas import tpu as pltpu
from jax.experimental.pallas import tpu_sc as plsc

N = 10000
D = 256
DH = 128
E = 160000
NL = 2
SLOPE = 0.2

NC, NS = 2, 16
NROWS = 10240
EPAD = 163840
NCHUNK = EPAD // 128
CPS_EDGE = NCHUNK // NS
CPS_DEG = NCHUNK // (NC * NS)
ROWS_PS = NROWS // NS

_f32 = jnp.float32
_i32 = jnp.int32


def _leaky(x):
    return jnp.where(x >= 0, x, SLOPE * x)


def _prep_body(wpd, wpm, owd, wvd, bvd, obd, owm, wvm, bvm, obm,
               adt, amt, bd, bm):
    dn_mm = (((1,), (0,)), ((), ()))
    md = lax.dot_general(owd[...], wvd[...], dn_mm, preferred_element_type=_f32)
    mm = lax.dot_general(owm[...], wvm[...], dn_mm, preferred_element_type=_f32)
    dn_tt = (((0,), (1,)), ((), ()))
    adt[...] = lax.dot_general(wpd[...], md, dn_tt, preferred_element_type=_f32)
    amt[...] = lax.dot_general(wpm[...], mm, dn_tt, preferred_element_type=_f32)
    dn_bt = (((1,), (1,)), ((), ()))
    bd[...] = lax.dot_general(bvd[...], owd[...], dn_bt,
                              preferred_element_type=_f32) + obd[...]
    bm[...] = lax.dot_general(bvm[...], owm[...], dn_bt,
                              preferred_element_type=_f32) + obm[...]


def _prep(wpd, wpm, owd, wvd, bvd, obd, owm, wvm, bvm, obm):
    kd, km = wpd.shape[1], wpm.shape[1]
    return pl.pallas_call(
        _prep_body,
        out_shape=[
            jax.ShapeDtypeStruct((kd, D), _f32),
            jax.ShapeDtypeStruct((km, D), _f32),
            jax.ShapeDtypeStruct((1, D), _f32),
            jax.ShapeDtypeStruct((1, D), _f32),
        ],
    )(wpd, wpm, owd, wvd, bvd, obd, owm, wvm, bvm, obm)


def _proj_body(d, m, nt, adt, amt, bd, bm, e2):
    dn = (((1,), (0,)), ((), ()))
    ed = lax.dot_general(d[...], adt[...], dn, preferred_element_type=_f32) + bd[...]
    em = lax.dot_general(m[...], amt[...], dn, preferred_element_type=_f32) + bm[...]
    e = jnp.where(nt[...] == 1, ed, em)
    e2[0] = e[:, :DH]
    e2[1] = e[:, DH:]


def _proj(d_features, m_features, nt2, adt, amt, bd, bm):
    blk = 1024
    kd, km = d_features.shape[1], m_features.shape[1]
    grid = (pl.cdiv(N, blk),)
    return pl.pallas_call(
        _proj_body,
        grid=grid,
        in_specs=[
            pl.BlockSpec((blk, kd), lambda i: (i, 0)),
            pl.BlockSpec((blk, km), lambda i: (i, 0)),
            pl.BlockSpec((blk, 1), lambda i: (i, 0)),
            pl.BlockSpec((kd, D), lambda i: (0, 0)),
            pl.BlockSpec((km, D), lambda i: (0, 0)),
            pl.BlockSpec((1, D), lambda i: (0, 0)),
            pl.BlockSpec((1, D), lambda i: (0, 0)),
        ],
        out_specs=pl.BlockSpec((NC, blk, DH), lambda i: (0, i, 0)),
        out_shape=jax.ShapeDtypeStruct((NC, N, DH), _f32),
    )(d_features, m_features, nt2, adt, amt, bd, bm)


def _dense_math(e2, m2, dg2, wst, wnt, wrt, bs, bn, br):
    x = jnp.concatenate([e2[0], e2[1]], axis=1)
    msg = jnp.concatenate([m2[0], m2[1]], axis=1)
    deg = dg2[0] + dg2[1]
    scale = 1.0 / (jnp.maximum(deg, 1.0) * jnp.maximum(deg, 1e-6))
    dn = (((1,), (0,)), ((), ()))
    self_h = _leaky(
        lax.dot_general(x, wst[...], dn, preferred_element_type=_f32) + bs[...])
    neigh_h = _leaky(
        lax.dot_general(msg * scale, wnt[...], dn,
                        preferred_element_type=_f32) + bn[...])
    return _leaky(
        lax.dot_general(self_h + neigh_h, wrt[...], dn,
                        preferred_element_type=_f32) + br[...])


def _dense_body(e2, m2, dg2, wst, wnt, wrt, bs, bn, br, y2):
    y = _dense_math(e2, m2, dg2, wst, wnt, wrt, bs, bn, br)
    y2[0] = y[:, :DH]
    y2[1] = y[:, DH:]


def _dense_last_body(e2, m2, dg2, wst, wnt, wrt, bs, bn, br, y):
    y[...] = _dense_math(e2, m2, dg2, wst, wnt, wrt, bs, bn, br)


def _dense(e2, m2, dg2, wst, wnt, wrt, bs, bn, br, last=False):
    blk = 1024
    grid = (pl.cdiv(N, blk),)
    fullspec = lambda a, b: pl.BlockSpec((a, b), lambda i: (0, 0))
    if last:
        out_specs = pl.BlockSpec((blk, D), lambda i: (i, 0))
        out_shape = jax.ShapeDtypeStruct((N, D), _f32)
        body = _dense_last_body
    else:
        out_specs = pl.BlockSpec((NC, blk, DH), lambda i: (0, i, 0))
        out_shape = jax.ShapeDtypeStruct((NC, N, DH), _f32)
        body = _dense_body
    return pl.pallas_call(
        body,
        grid=grid,
        in_specs=[
            pl.BlockSpec((NC, blk, DH), lambda i: (0, i, 0)),
            pl.BlockSpec((NC, blk, DH), lambda i: (0, i, 0)),
            pl.BlockSpec((NC, blk, 1), lambda i: (0, i, 0)),
            fullspec(D, D), fullspec(D, D), fullspec(D, D),
            fullspec(1, D), fullspec(1, D), fullspec(1, D),
        ],
        out_specs=out_specs,
        out_shape=out_shape,
    )(e2, m2, dg2, wst, wnt, wrt, bs, bn, br)


_MESH = plsc.VectorSubcoreMesh(core_axis_name="c", subcore_axis_name="s",
                               num_cores=NC, num_subcores=NS)


def _deg_body(esd, ones_h, zrow, deg2, idx_all, ones_v, dacc, sem_i, sem_s):
    c = lax.axis_index("c")
    s = lax.axis_index("s")
    for z in range(ROWS_PS // 128):
        pltpu.sync_copy(zrow, dacc.at[pl.ds(s * ROWS_PS + z * 128, 128)])
    base = c * (NCHUNK // NC) + s * CPS_DEG
    pltpu.async_copy(esd.at[pl.ds(base, CPS_DEG)], idx_all, sem_i)
    pltpu.sync_copy(ones_h, ones_v)
    pltpu.make_async_copy(esd.at[pl.ds(base, CPS_DEG)], idx_all, sem_i).wait()
    plsc.subcore_barrier()

    GRP = 8

    def deg_step(g, carry):
        for k in range(GRP):
            pltpu.async_copy(ones_v, dacc.at[idx_all.at[g * GRP + k].at[1]],
                             sem_s, add=True)
        for k in range(GRP):
            pltpu.make_async_copy(ones_v,
                                  dacc.at[idx_all.at[g * GRP + k].at[1]],
                                  sem_s).wait()
        return carry

    lax.fori_loop(0, CPS_DEG // GRP, deg_step, 0)

    plsc.subcore_barrier()
    pltpu.sync_copy(dacc.at[pl.ds(s * ROWS_PS, ROWS_PS)],
                    deg2.at[c].at[pl.ds(s * ROWS_PS, ROWS_PS)])


_deg = functools.partial(
    pl.kernel,
    out_type=jax.ShapeDtypeStruct((NC, NROWS, DH), _f32),
    mesh=_MESH,
    scratch_types=[
        pltpu.VMEM((CPS_DEG, 2, 128), _i32),
        pltpu.VMEM((128, DH), _f32),
        pltpu.VMEM_SHARED((NROWS, DH), _f32),
        pltpu.SemaphoreType.DMA,
        pltpu.SemaphoreType.DMA,
    ],
)(_deg_body)


def _train_body(st2, dt2, e2, si3, di3, tidx, rows_v, sem):
    c = lax.axis_index("c")
    s = lax.axis_index("s")

    def gather_train(t2, out):
        for t in range(2):
            pltpu.sync_copy(t2.at[s * 2 + t], tidx)
            pltpu.async_copy(e2.at[c].at[tidx.at[0]], rows_v, sem).wait()
            pltpu.sync_copy(rows_v,
                            out.at[pl.ds(s * 256 + t * 128, 128), c])

    gather_train(st2, si3)
    gather_train(dt2, di3)


_train = functools.partial(
    pl.kernel,
    out_type=(
        jax.ShapeDtypeStruct((4096, NC, DH), _f32),
        jax.ShapeDtypeStruct((4096, NC, DH), _f32),
    ),
    mesh=_MESH,
    scratch_types=[
        pltpu.VMEM((1, 128), _i32),
        pltpu.VMEM((128, DH), _f32),
        pltpu.SemaphoreType.DMA,
    ],
)(_train_body)


def _edge_body(esd, e2, zrow, m2, idx_all, rows_a, rows_b, acc,
               sem_a, sem_b, sem_sa, sem_sb, sem_i):
    c = lax.axis_index("c")
    s = lax.axis_index("s")
    STG = CPS_EDGE // 2
    pltpu.async_copy(esd.at[pl.ds(s * CPS_EDGE, STG)], idx_all, sem_i)
    for z in range(ROWS_PS // 128):
        pltpu.sync_copy(zrow, acc.at[pl.ds(s * ROWS_PS + z * 128, 128)])
    pltpu.make_async_copy(esd.at[pl.ds(s * CPS_EDGE, STG)], idx_all,
                          sem_i).wait()
    plsc.subcore_barrier()

    tab = e2.at[c]

    def run_stage(stg, carry):
        half = STG // 2
        pltpu.async_copy(tab.at[idx_all.at[0].at[0]], rows_a, sem_a)
        pltpu.async_copy(tab.at[idx_all.at[1].at[0]], rows_b, sem_b)

        def step(j, carry2):
            pltpu.make_async_copy(tab.at[idx_all.at[2 * j].at[0]],
                                  rows_a, sem_a).wait()
            pltpu.sync_copy(rows_a, acc.at[idx_all.at[2 * j].at[1]],
                            add=True)

            @pl.when(j < half - 1)
            def _():
                pltpu.async_copy(tab.at[idx_all.at[2 * j + 2].at[0]],
                                 rows_a, sem_a)

            pltpu.make_async_copy(tab.at[idx_all.at[2 * j + 1].at[0]],
                                  rows_b, sem_b).wait()
            pltpu.sync_copy(rows_b, acc.at[idx_all.at[2 * j + 1].at[1]],
                            add=True)

            @pl.when(j < half - 1)
            def _():
                pltpu.async_copy(tab.at[idx_all.at[2 * j + 3].at[0]],
                                 rows_b, sem_b)

            return carry2

        lax.fori_loop(0, half, step, 0)

        @pl.when(stg == 0)
        def _():
            pltpu.sync_copy(esd.at[pl.ds(s * CPS_EDGE + STG, STG)], idx_all)

        return carry

    lax.fori_loop(0, 2, run_stage, 0)
    plsc.subcore_barrier()
    pltpu.sync_copy(acc.at[pl.ds(s * ROWS_PS, ROWS_PS)],
                    m2.at[c].at[pl.ds(s * ROWS_PS, ROWS_PS)])


_edge = functools.partial(
    pl.kernel,
    out_type=jax.ShapeDtypeStruct((NC, NROWS, DH), _f32),
    mesh=_MESH,
    scratch_types=[
        pltpu.VMEM((CPS_EDGE // 2, 2, 128), _i32),
        pltpu.VMEM((128, DH), _f32),
        pltpu.VMEM((128, DH), _f32),
        pltpu.VMEM_SHARED((NROWS, DH), _f32),
        pltpu.SemaphoreType.DMA,
        pltpu.SemaphoreType.DMA,
        pltpu.SemaphoreType.DMA,
        pltpu.SemaphoreType.DMA,
        pltpu.SemaphoreType.DMA,
    ],
)(_edge_body)


def kernel(d_features, m_features, node_type, edge_index, src_train, dst_train,
           W_proj_d, W_proj_m, attn_d_in_w, attn_d_in_b, attn_d_out_w,
           attn_d_out_b, attn_m_in_w, attn_m_in_b, attn_m_out_w, attn_m_out_b,
           W_self, b_self, W_neigh, b_neigh, W_res, b_res):
    nt2 = node_type.astype(_i32).reshape(N, 1)
    src = edge_index[0].astype(_i32)
    dst = edge_index[1].astype(_i32)
    npad = EPAD - E
    pad_ar = jnp.arange(npad, dtype=_i32)
    src_p = jnp.concatenate([src, pad_ar % N])
    dst_p = jnp.concatenate([dst, N + pad_ar % (NROWS - N)])
    esd = jnp.stack([src_p.reshape(NCHUNK, 128), dst_p.reshape(NCHUNK, 128)],
                    axis=1)
    st2 = src_train.astype(_i32).reshape(32, 1, 128)
    dt2 = dst_train.astype(_i32).reshape(32, 1, 128)

    zrow = jnp.zeros((128, DH), _f32)
    ones_h = jnp.ones((128, DH), _f32)
    deg2 = _deg(esd, ones_h, zrow)
    dgs = deg2[:, :, :1]

    wvd = attn_d_in_w[2 * D:]
    bvd = attn_d_in_b[2 * D:].reshape(1, D)
    wvm = attn_m_in_w[2 * D:]
    bvm = attn_m_in_b[2 * D:].reshape(1, D)
    adt, amt, bd, bm = _prep(W_proj_d, W_proj_m,
                             attn_d_out_w, wvd, bvd, attn_d_out_b.reshape(1, D),
                             attn_m_out_w, wvm, bvm, attn_m_out_b.reshape(1, D))

    e2 = _proj(d_features, m_features, nt2, adt, amt, bd, bm)

    si3, di3 = _train(st2, dt2, e2)

    emb = None
    for l in range(NL):
        m2 = _edge(esd, e2, zrow)
        out = _dense(e2, m2, dgs,
                     W_self[l].T, W_neigh[l].T, W_res[l].T,
                     b_self[l].reshape(1, D), b_neigh[l].reshape(1, D),
                     b_res[l].reshape(1, D), last=(l == NL - 1))
        if l == NL - 1:
            emb = out
        else:
            e2 = out

    src_init = si3.reshape(4096, D)
    dst_init = di3.reshape(4096, D)
    return emb, src_init, dst_init
